# baseline (speedup 1.0000x reference)
import functools

import jax
import jax.numpy as jnp
from jax import lax
from jax.experimental import pallas as pl
from jax.experimental.pallas import tpu as pltpu
from jax.experimental.pallas import tpu_sc as plsc

BN = 2048

NC = 2
NS = 16
NW = NC * NS
CH = 128


def _vq_body(x_ref, et_ref, cnt_ref, idx_ref, idxo_ref, ncnt_ref):
    nb = pl.program_id(0)
    G = et_ref.shape[0]
    D = et_ref.shape[1]
    K = et_ref.shape[2]
    idx_cols = []
    idxo_cols = []
    cnt_rows = []
    for g in range(G):
        xg = x_ref[:, g * D:(g + 1) * D]
        et = et_ref[g]
        x_sq = jnp.sum(xg * xg, axis=-1, keepdims=True)
        e_sq = jnp.sum(et * et, axis=0, keepdims=True)
        cross = jnp.dot(xg, et, preferred_element_type=jnp.float32)
        d2 = jnp.maximum((x_sq - 2.0 * cross) + e_sq, 0.0)
        m = jnp.min(d2, axis=-1, keepdims=True)
        kiota = jax.lax.broadcasted_iota(jnp.int32, d2.shape, 1)
        idxc = jnp.min(jnp.where(d2 == m, kiota, K), axis=-1, keepdims=True)
        onehot = (kiota == idxc).astype(jnp.float32)
        idx_cols.append(idxc)
        idxo_cols.append(idxc + g * K)
        cnt_rows.append(jnp.sum(onehot, axis=0, keepdims=True))
    idx_ref[...] = jnp.concatenate(idx_cols, axis=1)
    idxo_ref[...] = jnp.concatenate(idxo_cols, axis=1)
    contrib = jnp.concatenate(cnt_rows, axis=0)

    @pl.when(nb == 0)
    def _():
        ncnt_ref[...] = cnt_ref[...] + contrib

    @pl.when(nb > 0)
    def _():
        ncnt_ref[...] = ncnt_ref[...] + contrib


def _gather_body(bpw, nch, table_hbm, idx_hbm, out_hbm, idx_v, rows_v, sem):
    wid = lax.axis_index("s") * NC + lax.axis_index("c")
    pltpu.sync_copy(idx_hbm.at[wid], idx_v)
    copies = []
    for j in range(nch):
        copies.append(
            pltpu.async_copy(table_hbm.at[idx_v.at[j]],
                             rows_v.at[pl.ds(j * CH, CH)], sem))
    for cp in copies:
        cp.wait()
    pltpu.sync_copy(rows_v, out_hbm.at[pl.ds(wid * bpw, bpw)])


def kernel(x, embeddings, count):
    BS, TPD, D = x.shape
    G, K, _ = embeddings.shape
    N = BS * TPD // G
    x2d = x.reshape(N, G * D)
    e_t = embeddings.transpose(0, 2, 1)
    grid = (N // BN,)
    idx_all, idx_off, ncnt = pl.pallas_call(
        _vq_body,
        grid=grid,
        in_specs=[
            pl.BlockSpec((BN, G * D), lambda i: (i, 0)),
            pl.BlockSpec((G, D, K), lambda i: (0, 0, 0)),
            pl.BlockSpec((G, K), lambda i: (0, 0)),
        ],
        out_specs=[
            pl.BlockSpec((BN, G), lambda i: (i, 0)),
            pl.BlockSpec((BN, G), lambda i: (i, 0)),
            pl.BlockSpec((G, K), lambda i: (0, 0)),
        ],
        out_shape=[
            jax.ShapeDtypeStruct((N, G), jnp.int32),
            jax.ShapeDtypeStruct((N, G), jnp.int32),
            jax.ShapeDtypeStruct((G, K), jnp.float32),
        ],
        compiler_params=pltpu.CompilerParams(
            dimension_semantics=("arbitrary",)),
    )(x2d, e_t, count)

    B = N * G
    bpw = B // NW
    nch = bpw // CH
    table = embeddings.reshape(G * K, D)
    idx3 = idx_off.reshape(NW, nch, CH)
    mesh = plsc.VectorSubcoreMesh(core_axis_name="c", subcore_axis_name="s")
    xq_flat = pl.kernel(
        functools.partial(_gather_body, bpw, nch),
        out_type=jax.ShapeDtypeStruct((B, D), jnp.float32),
        mesh=mesh,
        scratch_types=[
            pltpu.VMEM((nch, CH), jnp.int32),
            pltpu.VMEM((bpw, D), jnp.float32),
            pltpu.SemaphoreType.DMA,
        ],
        compiler_params=pltpu.CompilerParams(use_tc_tiling_on_sc=False),
    )(table, idx3)
    return xq_flat.reshape(BS, TPD, D), idx_all, ncnt


# speedup vs baseline: 1.0260x; 1.0260x over previous
import functools

import jax
import jax.numpy as jnp
from jax import lax
from jax.experimental import pallas as pl
from jax.experimental.pallas import tpu as pltpu
from jax.experimental.pallas import tpu_sc as plsc

BN = 2048

NC = 2
NS = 16
NW = NC * NS
CH = 128


def _vq_body(x_ref, et_ref, cnt_ref, idx_ref, idxo_ref, ncnt_ref):
    nb = pl.program_id(0)
    G = et_ref.shape[0]
    D = et_ref.shape[1]
    K = et_ref.shape[2]
    BNr = x_ref.shape[0]
    ones_row = jnp.ones((1, BNr), dtype=jnp.float32)
    idx_cols = []
    idxo_cols = []
    cnt_rows = []
    for g in range(G):
        xg = x_ref[:, g * D:(g + 1) * D]
        et = et_ref[g]
        x_sq = jnp.sum(xg * xg, axis=-1, keepdims=True)
        e_sq = jnp.sum(et * et, axis=0, keepdims=True)
        cross = jnp.dot(xg, et,
                        preferred_element_type=jnp.float32)
        d2 = jnp.maximum((x_sq - 2.0 * cross) + e_sq, 0.0)
        m = jnp.min(d2, axis=-1, keepdims=True)
        kiota = jax.lax.broadcasted_iota(jnp.int32, d2.shape, 1)
        idxc = jnp.min(jnp.where(d2 == m, kiota, K),
                       axis=-1, keepdims=True)
        onehot = (kiota == idxc).astype(jnp.float32)
        cnt = jnp.dot(ones_row, onehot,
                      preferred_element_type=jnp.float32)
        idx_cols.append(idxc)
        idxo_cols.append(idxc + g * K)
        cnt_rows.append(cnt)
    idx_ref[...] = jnp.concatenate(idx_cols, axis=1)
    idxo_ref[...] = jnp.concatenate(idxo_cols, axis=1)
    contrib = jnp.concatenate(cnt_rows, axis=0)

    @pl.when(nb == 0)
    def _():
        ncnt_ref[...] = cnt_ref[...] + contrib

    @pl.when(nb > 0)
    def _():
        ncnt_ref[...] = ncnt_ref[...] + contrib


def _gather_body(bpw, nch, table_hbm, idx_hbm, out_hbm, idx_v, rows_v, sem):
    wid = lax.axis_index("s") * NC + lax.axis_index("c")
    pltpu.sync_copy(idx_hbm.at[wid], idx_v)
    copies = []
    for j in range(nch):
        copies.append(
            pltpu.async_copy(table_hbm.at[idx_v.at[j]],
                             rows_v.at[pl.ds(j * CH, CH)], sem))
    for cp in copies:
        cp.wait()
    pltpu.sync_copy(rows_v, out_hbm.at[pl.ds(wid * bpw, bpw)])


def kernel(x, embeddings, count):
    BS, TPD, D = x.shape
    G, K, _ = embeddings.shape
    N = BS * TPD // G
    x2d = x.reshape(N, G * D)
    e_t = embeddings.transpose(0, 2, 1)
    grid = (N // BN,)
    idx_all, idx_off, ncnt = pl.pallas_call(
        _vq_body,
        grid=grid,
        in_specs=[
            pl.BlockSpec((BN, G * D), lambda i: (i, 0)),
            pl.BlockSpec((G, D, K), lambda i: (0, 0, 0)),
            pl.BlockSpec((G, K), lambda i: (0, 0)),
        ],
        out_specs=[
            pl.BlockSpec((BN, G), lambda i: (i, 0)),
            pl.BlockSpec((BN, G), lambda i: (i, 0)),
            pl.BlockSpec((G, K), lambda i: (0, 0)),
        ],
        out_shape=[
            jax.ShapeDtypeStruct((N, G), jnp.int32),
            jax.ShapeDtypeStruct((N, G), jnp.int32),
            jax.ShapeDtypeStruct((G, K), jnp.float32),
        ],
        compiler_params=pltpu.CompilerParams(
            dimension_semantics=("arbitrary",)),
    )(x2d, e_t, count)

    B = N * G
    bpw = B // NW
    nch = bpw // CH
    table = embeddings.reshape(G * K, D)
    idx3 = idx_off.reshape(NW, nch, CH)
    mesh = plsc.VectorSubcoreMesh(core_axis_name="c", subcore_axis_name="s")
    xq_flat = pl.kernel(
        functools.partial(_gather_body, bpw, nch),
        out_type=jax.ShapeDtypeStruct((B, D), jnp.float32),
        mesh=mesh,
        scratch_types=[
            pltpu.VMEM((nch, CH), jnp.int32),
            pltpu.VMEM((bpw, D), jnp.float32),
            pltpu.SemaphoreType.DMA,
        ],
        compiler_params=pltpu.CompilerParams(use_tc_tiling_on_sc=False),
    )(table, idx3)
    return xq_flat.reshape(BS, TPD, D), idx_all, ncnt


# speedup vs baseline: 1.1026x; 1.0746x over previous
import functools

import jax
import jax.numpy as jnp
from jax import lax
from jax.experimental import pallas as pl
from jax.experimental.pallas import tpu as pltpu
from jax.experimental.pallas import tpu_sc as plsc

BN = 2048

NC = 2
NS = 16
NW = NC * NS
CH = 128


def _vq_body(x_ref, et_ref, cnt_ref, idx_ref, idxo_ref, ncnt_ref):
    nb = pl.program_id(0)
    G = et_ref.shape[0]
    D = et_ref.shape[1]
    K = et_ref.shape[2]
    BNr = x_ref.shape[0]
    ones_row = jnp.ones((1, BNr), dtype=jnp.float32)
    idx_cols = []
    idxo_cols = []
    cnt_rows = []
    for g in range(G):
        xg = x_ref[:, g * D:(g + 1) * D]
        et = et_ref[g]
        x_sq = jnp.sum(xg * xg, axis=-1, keepdims=True)
        e_sq = jnp.sum(et * et, axis=0, keepdims=True)
        cross = jnp.dot(xg, et,
                        preferred_element_type=jnp.float32)
        d2 = jnp.maximum((x_sq - 2.0 * cross) + e_sq, 0.0)
        m = jnp.min(d2, axis=-1, keepdims=True)
        kiota = jax.lax.broadcasted_iota(
            jnp.int32, d2.shape, 1).astype(jnp.float32)
        idx_f = jnp.min(jnp.where(d2 == m, kiota, float(K)),
                        axis=-1, keepdims=True)
        onehot = jnp.where(kiota == idx_f, 1.0, 0.0)
        cnt = jnp.dot(ones_row, onehot,
                      preferred_element_type=jnp.float32)
        idxc = idx_f.astype(jnp.int32)
        idx_cols.append(idxc)
        idxo_cols.append(idxc + g * K)
        cnt_rows.append(cnt)
    idx_ref[...] = jnp.concatenate(idx_cols, axis=1)
    idxo_ref[...] = jnp.concatenate(idxo_cols, axis=1)
    contrib = jnp.concatenate(cnt_rows, axis=0)

    @pl.when(nb == 0)
    def _():
        ncnt_ref[...] = cnt_ref[...] + contrib

    @pl.when(nb > 0)
    def _():
        ncnt_ref[...] = ncnt_ref[...] + contrib


def _gather_body(bpw, nch, table_hbm, idx_hbm, out_hbm, idx_v, rows_v, sem):
    wid = lax.axis_index("s") * NC + lax.axis_index("c")
    pltpu.sync_copy(idx_hbm.at[wid], idx_v)
    copies = []
    for j in range(nch):
        copies.append(
            pltpu.async_copy(table_hbm.at[idx_v.at[j]],
                             rows_v.at[pl.ds(j * CH, CH)], sem))
    for cp in copies:
        cp.wait()
    pltpu.sync_copy(rows_v, out_hbm.at[pl.ds(wid * bpw, bpw)])


def kernel(x, embeddings, count):
    BS, TPD, D = x.shape
    G, K, _ = embeddings.shape
    N = BS * TPD // G
    x2d = x.reshape(N, G * D)
    e_t = embeddings.transpose(0, 2, 1)
    grid = (N // BN,)
    idx_all, idx_off, ncnt = pl.pallas_call(
        _vq_body,
        grid=grid,
        in_specs=[
            pl.BlockSpec((BN, G * D), lambda i: (i, 0)),
            pl.BlockSpec((G, D, K), lambda i: (0, 0, 0)),
            pl.BlockSpec((G, K), lambda i: (0, 0)),
        ],
        out_specs=[
            pl.BlockSpec((BN, G), lambda i: (i, 0)),
            pl.BlockSpec((BN, G), lambda i: (i, 0)),
            pl.BlockSpec((G, K), lambda i: (0, 0)),
        ],
        out_shape=[
            jax.ShapeDtypeStruct((N, G), jnp.int32),
            jax.ShapeDtypeStruct((N, G), jnp.int32),
            jax.ShapeDtypeStruct((G, K), jnp.float32),
        ],
        compiler_params=pltpu.CompilerParams(
            dimension_semantics=("arbitrary",)),
    )(x2d, e_t, count)

    B = N * G
    bpw = B // NW
    nch = bpw // CH
    table = embeddings.reshape(G * K, D)
    idx3 = idx_off.reshape(NW, nch, CH)
    mesh = plsc.VectorSubcoreMesh(core_axis_name="c", subcore_axis_name="s")
    xq_flat = pl.kernel(
        functools.partial(_gather_body, bpw, nch),
        out_type=jax.ShapeDtypeStruct((B, D), jnp.float32),
        mesh=mesh,
        scratch_types=[
            pltpu.VMEM((nch, CH), jnp.int32),
            pltpu.VMEM((bpw, D), jnp.float32),
            pltpu.SemaphoreType.DMA,
        ],
        compiler_params=pltpu.CompilerParams(use_tc_tiling_on_sc=False),
    )(table, idx3)
    return xq_flat.reshape(BS, TPD, D), idx_all, ncnt


# speedup vs baseline: 1.1510x; 1.0439x over previous
import functools

import jax
import jax.numpy as jnp
from jax import lax
from jax.experimental import pallas as pl
from jax.experimental.pallas import tpu as pltpu
from jax.experimental.pallas import tpu_sc as plsc

BN = 4096

NC = 2
NS = 16
NW = NC * NS
CH = 128


def _vq_body(x_ref, et_ref, cnt_ref, idx_ref, idxo_ref, ncnt_ref):
    nb = pl.program_id(0)
    G = et_ref.shape[0]
    D = et_ref.shape[1]
    K = et_ref.shape[2]
    BNr = x_ref.shape[0]
    ones_row = jnp.ones((1, BNr), dtype=jnp.float32)
    idx_cols = []
    idxo_cols = []
    cnt_rows = []
    for g in range(G):
        xg = x_ref[:, g * D:(g + 1) * D]
        et = et_ref[g]
        x_sq = jnp.sum(xg * xg, axis=-1, keepdims=True)
        e_sq = jnp.sum(et * et, axis=0, keepdims=True)
        cross = jnp.dot(xg, et,
                        preferred_element_type=jnp.float32)
        d2 = (x_sq - 2.0 * cross) + e_sq
        m = jnp.min(d2, axis=-1, keepdims=True)
        kiota = jax.lax.broadcasted_iota(
            jnp.int32, d2.shape, 1).astype(jnp.float32)
        idx_f = jnp.min(jnp.where(d2 == m, kiota, float(K)),
                        axis=-1, keepdims=True)
        onehot = jnp.where(kiota == idx_f, 1.0, 0.0)
        cnt = jnp.dot(ones_row, onehot,
                      preferred_element_type=jnp.float32)
        idxc = idx_f.astype(jnp.int32)
        idx_cols.append(idxc)
        idxo_cols.append(idxc + g * K)
        cnt_rows.append(cnt)
    idx_ref[...] = jnp.concatenate(idx_cols, axis=1)
    idxo_ref[...] = jnp.concatenate(idxo_cols, axis=1)
    contrib = jnp.concatenate(cnt_rows, axis=0)

    @pl.when(nb == 0)
    def _():
        ncnt_ref[...] = cnt_ref[...] + contrib

    @pl.when(nb > 0)
    def _():
        ncnt_ref[...] = ncnt_ref[...] + contrib


def _gather_body(bpw, nch, table_hbm, idx_hbm, out_hbm, idx_v, rows_v, sem):
    wid = lax.axis_index("s") * NC + lax.axis_index("c")
    pltpu.sync_copy(idx_hbm.at[wid], idx_v)
    copies = []
    for j in range(nch):
        copies.append(
            pltpu.async_copy(table_hbm.at[idx_v.at[j]],
                             rows_v.at[pl.ds(j * CH, CH)], sem))
    for cp in copies:
        cp.wait()
    pltpu.sync_copy(rows_v, out_hbm.at[pl.ds(wid * bpw, bpw)])


def kernel(x, embeddings, count):
    BS, TPD, D = x.shape
    G, K, _ = embeddings.shape
    N = BS * TPD // G
    x2d = x.reshape(N, G * D)
    e_t = embeddings.transpose(0, 2, 1)
    grid = (N // BN,)
    idx_all, idx_off, ncnt = pl.pallas_call(
        _vq_body,
        grid=grid,
        in_specs=[
            pl.BlockSpec((BN, G * D), lambda i: (i, 0)),
            pl.BlockSpec((G, D, K), lambda i: (0, 0, 0)),
            pl.BlockSpec((G, K), lambda i: (0, 0)),
        ],
        out_specs=[
            pl.BlockSpec((BN, G), lambda i: (i, 0)),
            pl.BlockSpec((BN, G), lambda i: (i, 0)),
            pl.BlockSpec((G, K), lambda i: (0, 0)),
        ],
        out_shape=[
            jax.ShapeDtypeStruct((N, G), jnp.int32),
            jax.ShapeDtypeStruct((N, G), jnp.int32),
            jax.ShapeDtypeStruct((G, K), jnp.float32),
        ],
        compiler_params=pltpu.CompilerParams(
            dimension_semantics=("arbitrary",)),
    )(x2d, e_t, count)

    B = N * G
    bpw = B // NW
    nch = bpw // CH
    table = embeddings.reshape(G * K, D)
    idx3 = idx_off.reshape(NW, nch, CH)
    mesh = plsc.VectorSubcoreMesh(core_axis_name="c", subcore_axis_name="s")
    xq_flat = pl.kernel(
        functools.partial(_gather_body, bpw, nch),
        out_type=jax.ShapeDtypeStruct((B, D), jnp.float32),
        mesh=mesh,
        scratch_types=[
            pltpu.VMEM((nch, CH), jnp.int32),
            pltpu.VMEM((bpw, D), jnp.float32),
            pltpu.SemaphoreType.DMA,
        ],
        compiler_params=pltpu.CompilerParams(use_tc_tiling_on_sc=False),
    )(table, idx3)
    return xq_flat.reshape(BS, TPD, D), idx_all, ncnt
